# baseline (speedup 1.0000x reference)
import functools

import jax
import jax.numpy as jnp
from jax import lax
from jax.experimental import pallas as pl
from jax.experimental.pallas import tpu as pltpu
from jax.experimental.pallas import tpu_sc as plsc

N, E, D, DE, H = 10000, 320000, 128, 4, 128

NC, NS = 2, 16
NW = NC * NS
EPW = E // NW
CHUNK = 80
NCHUNK = EPW // CHUNK
NBUF = 4
GROUPS = (NCHUNK - 1) // NBUF
RPT = 624
RTAIL = N - NS * RPT


def _y_body(x_ref, w_ref, b_ref, o_ref):
    o_ref[...] = jnp.dot(x_ref[...], w_ref[...],
                         preferred_element_type=jnp.float32) + b_ref[...]


def _upd_body(x_ref, a_ref, wx_ref, wa_ref, b_ref, o_ref):
    agg = a_ref[0] + a_ref[1]
    h = (jnp.dot(x_ref[...], wx_ref[...], preferred_element_type=jnp.float32)
         + jnp.dot(agg, wa_ref[...], preferred_element_type=jnp.float32)
         + b_ref[...])
    o_ref[...] = jnp.maximum(h, 0.0) + x_ref[...]


def _tc_y(x, w1, bm):
    bn = 2000
    return pl.pallas_call(
        _y_body,
        grid=(N // bn,),
        in_specs=[pl.BlockSpec((bn, D), lambda i: (i, 0)),
                  pl.BlockSpec((D, H), lambda i: (0, 0)),
                  pl.BlockSpec((1, H), lambda i: (0, 0))],
        out_specs=pl.BlockSpec((bn, H), lambda i: (i, 0)),
        out_shape=jax.ShapeDtypeStruct((N, H), jnp.float32),
    )(x, w1, bm)


def _tc_update(x, agg_p, wx, wa, bu):
    bn = 2000
    return pl.pallas_call(
        _upd_body,
        grid=(N // bn,),
        in_specs=[pl.BlockSpec((bn, D), lambda i: (i, 0)),
                  pl.BlockSpec((2, bn, H), lambda i: (0, i, 0)),
                  pl.BlockSpec((D, D), lambda i: (0, 0)),
                  pl.BlockSpec((H, D), lambda i: (0, 0)),
                  pl.BlockSpec((1, D), lambda i: (0, 0))],
        out_specs=pl.BlockSpec((bn, D), lambda i: (i, 0)),
        out_shape=jax.ShapeDtypeStruct((N, D), jnp.float32),
    )(x, agg_p, wx, wa, bu)


def _sc_edges(src, dst, ea_flat, w2, y):
    mesh = plsc.VectorSubcoreMesh(core_axis_name="c", subcore_axis_name="s")

    scratch = (
        [pltpu.VMEM((CHUNK,), jnp.int32) for _ in range(2 * NBUF)]
        + [pltpu.VMEM((DE * CHUNK,), jnp.float32) for _ in range(NBUF)]
        + [pltpu.VMEM((NBUF * CHUNK, H), jnp.float32)]
        + [pltpu.VMEM((DE, H), jnp.float32)]
        + [pltpu.SemaphoreType.DMA for _ in range(3 * NBUF)]
        + [pltpu.VMEM_SHARED((N, H), jnp.float32)]
    )

    @functools.partial(
        pl.kernel,
        mesh=mesh,
        out_type=jax.ShapeDtypeStruct((NC, N, H), jnp.float32),
        scratch_types=scratch,
    )
    def k(src_hbm, dst_hbm, ea_hbm, w2_hbm, y_hbm, out_hbm, *rest):
        idx_s = rest[0:NBUF]
        idx_d = rest[NBUF:2 * NBUF]
        abuf = rest[2 * NBUF:3 * NBUF]
        mbuf = rest[3 * NBUF]
        w2buf = rest[3 * NBUF + 1]
        lsem = rest[3 * NBUF + 2:3 * NBUF + 2 + NBUF]
        gsem = rest[3 * NBUF + 2 + NBUF:3 * NBUF + 2 + 2 * NBUF]
        ssem = rest[3 * NBUF + 2 + 2 * NBUF:3 * NBUF + 2 + 3 * NBUF]
        acc = rest[-1]

        cid = lax.axis_index("c")
        sid = lax.axis_index("s")
        wid = cid * NS + sid
        ebase = wid * EPW
        rbase = sid * RPT

        def msl(j):
            return mbuf.at[pl.ds(j * CHUNK, CHUNK)]

        pltpu.sync_copy(w2_hbm, w2buf)

        @plsc.parallel_loop(0, NBUF * CHUNK, unroll=4)
        def _(i):
            for q in range(H // 16):
                mbuf[i, pl.ds(q * 16, 16)] = jnp.zeros((16,), jnp.float32)
        pltpu.sync_copy(mbuf.at[pl.ds(0, NBUF * CHUNK)],
                        acc.at[pl.ds(rbase, NBUF * CHUNK)])
        pltpu.sync_copy(mbuf.at[pl.ds(0, RPT - NBUF * CHUNK)],
                        acc.at[pl.ds(rbase + NBUF * CHUNK,
                                     RPT - NBUF * CHUNK)])

        @pl.when(sid == 0)
        def _():
            pltpu.sync_copy(mbuf.at[pl.ds(0, RTAIL)],
                            acc.at[pl.ds(NS * RPT, RTAIL)])
        plsc.subcore_barrier()

        def issue_loads(c, j):
            base = ebase + c * CHUNK
            pltpu.async_copy(src_hbm.at[pl.ds(base, CHUNK)], idx_s[j], lsem[j])
            pltpu.async_copy(dst_hbm.at[pl.ds(base, CHUNK)], idx_d[j], lsem[j])
            pltpu.async_copy(ea_hbm.at[pl.ds(base * DE, CHUNK * DE)], abuf[j],
                             lsem[j])

        def wait_loads(c, j):
            base = ebase + c * CHUNK
            pltpu.make_async_copy(src_hbm.at[pl.ds(base, CHUNK)], idx_s[j],
                                  lsem[j]).wait()
            pltpu.make_async_copy(dst_hbm.at[pl.ds(base, CHUNK)], idx_d[j],
                                  lsem[j]).wait()
            pltpu.make_async_copy(ea_hbm.at[pl.ds(base * DE, CHUNK * DE)],
                                  abuf[j], lsem[j]).wait()

        def edge_math(j):
            br = j * CHUNK
            aj = abuf[j]

            @plsc.parallel_loop(0, CHUNK // 4)
            def _(e4):
                avec = aj[pl.ds(e4 * 16, 16)]
                ab = [jnp.broadcast_to(avec[i], (16,)) for i in range(16)]
                for q in range(H // 16):
                    sl = pl.ds(q * 16, 16)
                    w2q = [w2buf[kk, sl] for kk in range(DE)]
                    for sub in range(4):
                        row = br + e4 * 4 + sub
                        v = mbuf[row, sl]
                        for kk in range(DE):
                            v = v + ab[4 * sub + kk] * w2q[kk]
                        mbuf[row, sl] = jnp.maximum(v, 0.0)

        for j in range(NBUF):
            issue_loads(j, j)

        def group(i, _):
            gathers = []
            for j in range(NBUF):
                wait_loads(i * NBUF + j, j)
                gathers.append(
                    pltpu.async_copy(y_hbm.at[idx_s[j]], msl(j), gsem[j]))
            scatters = []
            for j in range(NBUF):
                gathers[j].wait()
                edge_math(j)
                scatters.append(
                    pltpu.async_copy(msl(j), acc.at[idx_d[j]], ssem[j],
                                     add=True))
            for j in range(NBUF):
                scatters[j].wait()

                @pl.when(i < GROUPS - 1)
                def _(j=j):
                    issue_loads((i + 1) * NBUF + j, j)

                if j == 0:
                    @pl.when(i == GROUPS - 1)
                    def _():
                        issue_loads(NCHUNK - 1, 0)
            return 0
        lax.fori_loop(0, GROUPS, group, 0)

        wait_loads(NCHUNK - 1, 0)
        pltpu.async_copy(y_hbm.at[idx_s[0]], msl(0), gsem[0]).wait()
        edge_math(0)
        pltpu.async_copy(msl(0), acc.at[idx_d[0]], ssem[0], add=True).wait()

        plsc.subcore_barrier()
        pltpu.sync_copy(acc.at[pl.ds(rbase, RPT)],
                        out_hbm.at[cid, pl.ds(rbase, RPT)])

        @pl.when(sid == 0)
        def _():
            pltpu.sync_copy(acc.at[pl.ds(NS * RPT, RTAIL)],
                            out_hbm.at[cid, pl.ds(NS * RPT, RTAIL)])

    return k(src, dst, ea_flat, w2, y)


def kernel(x, edge_index, edge_attr, W_msg, b_msg, W_upd, b_upd):
    w1 = W_msg[:D]
    w2 = W_msg[D:]
    wx = W_upd[:D]
    wa = W_upd[D:]
    bm = b_msg.reshape(1, H)
    bu = b_upd.reshape(1, D)
    ea_flat = edge_attr.reshape(E * DE)

    y = _tc_y(x, w1, bm)
    agg_p = _sc_edges(edge_index[0], edge_index[1], ea_flat, w2, y)
    return _tc_update(x, agg_p, wx, wa, bu)


# speedup vs baseline: 1.0602x; 1.0602x over previous
import functools

import jax
import jax.numpy as jnp
from jax import lax
from jax.experimental import pallas as pl
from jax.experimental.pallas import tpu as pltpu
from jax.experimental.pallas import tpu_sc as plsc

N, E, D, DE, H = 10000, 320000, 128, 4, 128

NC, NS = 2, 16
NW = NC * NS
EPW = E // NW
CHUNK = 80
NCHUNK = EPW // CHUNK
NBUF = 4
GROUPS = (NCHUNK - 1) // NBUF
RPT = 624
RTAIL = N - NS * RPT


def _y_body(x_ref, w_ref, b_ref, o_ref):
    o_ref[...] = jnp.dot(x_ref[...], w_ref[...],
                         preferred_element_type=jnp.float32) + b_ref[...]


def _upd_body(x_ref, a_ref, wx_ref, wa_ref, b_ref, o_ref):
    agg = a_ref[0] + a_ref[1]
    h = (jnp.dot(x_ref[...], wx_ref[...], preferred_element_type=jnp.float32)
         + jnp.dot(agg, wa_ref[...], preferred_element_type=jnp.float32)
         + b_ref[...])
    o_ref[...] = jnp.maximum(h, 0.0) + x_ref[...]


def _tc_y(x, w1, bm):
    bn = 2000
    return pl.pallas_call(
        _y_body,
        grid=(N // bn,),
        in_specs=[pl.BlockSpec((bn, D), lambda i: (i, 0)),
                  pl.BlockSpec((D, H), lambda i: (0, 0)),
                  pl.BlockSpec((1, H), lambda i: (0, 0))],
        out_specs=pl.BlockSpec((bn, H), lambda i: (i, 0)),
        out_shape=jax.ShapeDtypeStruct((N, H), jnp.float32),
    )(x, w1, bm)


def _tc_update(x, agg_p, wx, wa, bu):
    bn = 2000
    return pl.pallas_call(
        _upd_body,
        grid=(N // bn,),
        in_specs=[pl.BlockSpec((bn, D), lambda i: (i, 0)),
                  pl.BlockSpec((2, bn, H), lambda i: (0, i, 0)),
                  pl.BlockSpec((D, D), lambda i: (0, 0)),
                  pl.BlockSpec((H, D), lambda i: (0, 0)),
                  pl.BlockSpec((1, D), lambda i: (0, 0))],
        out_specs=pl.BlockSpec((bn, D), lambda i: (i, 0)),
        out_shape=jax.ShapeDtypeStruct((N, D), jnp.float32),
    )(x, agg_p, wx, wa, bu)


def _sc_edges(src, dst, ea_flat, w2, y):
    mesh = plsc.VectorSubcoreMesh(core_axis_name="c", subcore_axis_name="s")

    scratch = (
        [pltpu.VMEM((CHUNK,), jnp.int32) for _ in range(2 * NBUF)]
        + [pltpu.VMEM((DE * CHUNK,), jnp.float32) for _ in range(NBUF)]
        + [pltpu.VMEM((NBUF * CHUNK, H), jnp.float32)]
        + [pltpu.VMEM((DE, H), jnp.float32)]
        + [pltpu.SemaphoreType.DMA for _ in range(3 * NBUF)]
        + [pltpu.VMEM_SHARED((N, H), jnp.float32)]
    )

    @functools.partial(
        pl.kernel,
        mesh=mesh,
        out_type=jax.ShapeDtypeStruct((NC, N, H), jnp.float32),
        scratch_types=scratch,
    )
    def k(src_hbm, dst_hbm, ea_hbm, w2_hbm, y_hbm, out_hbm, *rest):
        idx_s = rest[0:NBUF]
        idx_d = rest[NBUF:2 * NBUF]
        abuf = rest[2 * NBUF:3 * NBUF]
        mbuf = rest[3 * NBUF]
        w2buf = rest[3 * NBUF + 1]
        lsem = rest[3 * NBUF + 2:3 * NBUF + 2 + NBUF]
        gsem = rest[3 * NBUF + 2 + NBUF:3 * NBUF + 2 + 2 * NBUF]
        ssem = rest[3 * NBUF + 2 + 2 * NBUF:3 * NBUF + 2 + 3 * NBUF]
        acc = rest[-1]

        cid = lax.axis_index("c")
        sid = lax.axis_index("s")
        wid = cid * NS + sid
        ebase = wid * EPW
        rbase = sid * RPT

        def msl(j):
            return mbuf.at[pl.ds(j * CHUNK, CHUNK)]

        pltpu.sync_copy(w2_hbm, w2buf)
        w2v = [[w2buf[kk, pl.ds(q * 16, 16)] for q in range(H // 16)]
               for kk in range(DE)]

        @plsc.parallel_loop(0, NBUF * CHUNK, unroll=4)
        def _(i):
            for q in range(H // 16):
                mbuf[i, pl.ds(q * 16, 16)] = jnp.zeros((16,), jnp.float32)
        pltpu.sync_copy(mbuf.at[pl.ds(0, NBUF * CHUNK)],
                        acc.at[pl.ds(rbase, NBUF * CHUNK)])
        pltpu.sync_copy(mbuf.at[pl.ds(0, RPT - NBUF * CHUNK)],
                        acc.at[pl.ds(rbase + NBUF * CHUNK,
                                     RPT - NBUF * CHUNK)])

        @pl.when(sid == 0)
        def _():
            pltpu.sync_copy(mbuf.at[pl.ds(0, RTAIL)],
                            acc.at[pl.ds(NS * RPT, RTAIL)])
        plsc.subcore_barrier()

        def issue_loads(c, j):
            base = ebase + c * CHUNK
            pltpu.async_copy(src_hbm.at[pl.ds(base, CHUNK)], idx_s[j], lsem[j])
            pltpu.async_copy(dst_hbm.at[pl.ds(base, CHUNK)], idx_d[j], lsem[j])
            pltpu.async_copy(ea_hbm.at[pl.ds(base * DE, CHUNK * DE)], abuf[j],
                             lsem[j])

        def wait_loads(c, j):
            base = ebase + c * CHUNK
            pltpu.make_async_copy(src_hbm.at[pl.ds(base, CHUNK)], idx_s[j],
                                  lsem[j]).wait()
            pltpu.make_async_copy(dst_hbm.at[pl.ds(base, CHUNK)], idx_d[j],
                                  lsem[j]).wait()
            pltpu.make_async_copy(ea_hbm.at[pl.ds(base * DE, CHUNK * DE)],
                                  abuf[j], lsem[j]).wait()

        def edge_math(j):
            br = j * CHUNK
            aj = abuf[j]

            @plsc.parallel_loop(0, CHUNK // 4)
            def _(e4):
                avec = aj[pl.ds(e4 * 16, 16)]
                for sub in range(4):
                    row = br + e4 * 4 + sub
                    ab = [jnp.broadcast_to(avec[4 * sub + kk], (16,))
                          for kk in range(DE)]
                    for q in range(H // 16):
                        sl = pl.ds(q * 16, 16)
                        v = mbuf[row, sl]
                        for kk in range(DE):
                            v = v + ab[kk] * w2v[kk][q]
                        mbuf[row, sl] = jnp.maximum(v, 0.0)

        for j in range(NBUF):
            issue_loads(j, j)

        def group(i, _):
            gathers = []
            for j in range(NBUF):
                wait_loads(i * NBUF + j, j)
                gathers.append(
                    pltpu.async_copy(y_hbm.at[idx_s[j]], msl(j), gsem[j]))
            scatters = []
            for j in range(NBUF):
                gathers[j].wait()
                edge_math(j)
                scatters.append(
                    pltpu.async_copy(msl(j), acc.at[idx_d[j]], ssem[j],
                                     add=True))
            for j in range(NBUF):
                scatters[j].wait()

                @pl.when(i < GROUPS - 1)
                def _(j=j):
                    issue_loads((i + 1) * NBUF + j, j)

                if j == 0:
                    @pl.when(i == GROUPS - 1)
                    def _():
                        issue_loads(NCHUNK - 1, 0)
            return 0
        lax.fori_loop(0, GROUPS, group, 0)

        wait_loads(NCHUNK - 1, 0)
        pltpu.async_copy(y_hbm.at[idx_s[0]], msl(0), gsem[0]).wait()
        edge_math(0)
        pltpu.async_copy(msl(0), acc.at[idx_d[0]], ssem[0], add=True).wait()

        plsc.subcore_barrier()
        pltpu.sync_copy(acc.at[pl.ds(rbase, RPT)],
                        out_hbm.at[cid, pl.ds(rbase, RPT)])

        @pl.when(sid == 0)
        def _():
            pltpu.sync_copy(acc.at[pl.ds(NS * RPT, RTAIL)],
                            out_hbm.at[cid, pl.ds(NS * RPT, RTAIL)])

    return k(src, dst, ea_flat, w2, y)


def kernel(x, edge_index, edge_attr, W_msg, b_msg, W_upd, b_upd):
    w1 = W_msg[:D]
    w2 = W_msg[D:]
    wx = W_upd[:D]
    wa = W_upd[D:]
    bm = b_msg.reshape(1, H)
    bu = b_upd.reshape(1, D)
    ea_flat = edge_attr.reshape(E * DE)

    y = _tc_y(x, w1, bm)
    agg_p = _sc_edges(edge_index[0], edge_index[1], ea_flat, w2, y)
    return _tc_update(x, agg_p, wx, wa, bu)
